# Initial kernel scaffold; baseline (speedup 1.0000x reference)
import jax
import jax.numpy as jnp
from jax.experimental import pallas as pl


def kernel(latent, labels, centers):
    raise NotImplementedError("write your pallas kernel here")


# speedup vs baseline: 1.8574x; 1.8574x over previous
import functools

import jax
import jax.numpy as jnp
from jax import lax
from jax.experimental import pallas as pl
from jax.experimental.pallas import tpu as pltpu
from jax.experimental.pallas import tpu_sc as plsc

_B = 16384
_D = 128
_NC = 2
_NS = 16
_NW = _NC * _NS
_RPW = _B // _NW
_CH = 128
_NCH = _RPW // _CH
_LANES = 16
_JV = _D // _LANES


def _sc_body(latent_hbm, labels_hbm, centers_hbm, out_hbm,
             lab_v, lat_v, cen_v, res_v, lat_sem, gat_sem, out_sem):
    wid = lax.axis_index("s") * _NC + lax.axis_index("c")
    pltpu.sync_copy(labels_hbm.at[wid], lab_v)

    accs = tuple(jnp.zeros((_LANES,), jnp.float32) for _ in range(_JV))
    for ch in range(_NCH):
        row0 = wid * _RPW + ch * _CH
        lat_copy = pltpu.async_copy(
            latent_hbm.at[pl.ds(row0, _CH)], lat_v, lat_sem)
        gat_copy = pltpu.async_copy(
            centers_hbm.at[lab_v.at[ch]], cen_v, gat_sem)
        lat_copy.wait()
        gat_copy.wait()

        def row_body(r, accs):
            new = []
            for j in range(_JV):
                lt = lat_v[r, pl.ds(j * _LANES, _LANES)]
                cn = cen_v[r, pl.ds(j * _LANES, _LANES)]
                d = lt - cn
                new.append(accs[j] + d * d)
            return tuple(new)

        accs = lax.fori_loop(0, _CH, row_body, accs)

    total = accs[0]
    for j in range(1, _JV):
        total = total + accs[j]
    res_v[...] = total
    pltpu.async_copy(res_v, out_hbm.at[wid], out_sem).wait()


@jax.jit
def _center_loss_partials(latent, labels3d, centers):
    mesh = plsc.VectorSubcoreMesh(core_axis_name="c", subcore_axis_name="s")
    run = functools.partial(
        pl.kernel,
        out_type=jax.ShapeDtypeStruct((_NW, _LANES), jnp.float32),
        mesh=mesh,
        scratch_types=[
            pltpu.VMEM((_NCH, _CH), jnp.int32),
            pltpu.VMEM((_CH, _D), jnp.float32),
            pltpu.VMEM((_CH, _D), jnp.float32),
            pltpu.VMEM((_LANES,), jnp.float32),
            pltpu.SemaphoreType.DMA,
            pltpu.SemaphoreType.DMA,
            pltpu.SemaphoreType.DMA,
        ],
    )(_sc_body)
    return run(latent, labels3d, centers)


def kernel(latent, labels, centers):
    labels3d = labels.astype(jnp.int32).reshape(_NW, _NCH, _CH)
    partials = _center_loss_partials(latent, labels3d, centers)
    return jnp.sum(partials) / jnp.float32(_B)


# speedup vs baseline: 2.0360x; 1.0961x over previous
import functools

import jax
import jax.numpy as jnp
from jax import lax
from jax.experimental import pallas as pl
from jax.experimental.pallas import tpu as pltpu
from jax.experimental.pallas import tpu_sc as plsc

_B = 16384
_D = 128
_NC = 2
_NS = 16
_NW = _NC * _NS
_RPW = _B // _NW
_CH = 128
_NCH = _RPW // _CH
_LANES = 16
_JV = _D // _LANES


def _sc_body(latent_hbm, labels_hbm, centers_hbm, out_hbm,
             lab_v, lat_v0, lat_v1, cen_v0, cen_v1, res_v,
             lat_sem0, lat_sem1, gat_sem0, gat_sem1, out_sem):
    wid = lax.axis_index("s") * _NC + lax.axis_index("c")
    pltpu.sync_copy(labels_hbm.at[wid], lab_v)

    lat_bufs = (lat_v0, lat_v1)
    cen_bufs = (cen_v0, cen_v1)
    lat_sems = (lat_sem0, lat_sem1)
    gat_sems = (gat_sem0, gat_sem1)

    def start(ch):
        b = ch % 2
        row0 = wid * _RPW + ch * _CH
        lat = pltpu.async_copy(
            latent_hbm.at[pl.ds(row0, _CH)], lat_bufs[b], lat_sems[b])
        gat = pltpu.async_copy(
            centers_hbm.at[lab_v.at[ch]], cen_bufs[b], gat_sems[b])
        return lat, gat

    accs = tuple(jnp.zeros((_LANES,), jnp.float32) for _ in range(_JV))
    pend = start(0)
    for ch in range(_NCH):
        b = ch % 2
        lat_copy, gat_copy = pend
        if ch + 1 < _NCH:
            pend = start(ch + 1)
        lat_copy.wait()
        gat_copy.wait()
        lat_v, cen_v = lat_bufs[b], cen_bufs[b]

        def row_body(r, accs):
            new = []
            for j in range(_JV):
                lt = lat_v[r, pl.ds(j * _LANES, _LANES)]
                cn = cen_v[r, pl.ds(j * _LANES, _LANES)]
                d = lt - cn
                new.append(accs[j] + d * d)
            return tuple(new)

        accs = lax.fori_loop(0, _CH, row_body, accs)

    total = accs[0]
    for j in range(1, _JV):
        total = total + accs[j]
    res_v[...] = total
    pltpu.async_copy(res_v, out_hbm.at[wid], out_sem).wait()


@jax.jit
def _center_loss_partials(latent, labels3d, centers):
    mesh = plsc.VectorSubcoreMesh(core_axis_name="c", subcore_axis_name="s")
    run = functools.partial(
        pl.kernel,
        out_type=jax.ShapeDtypeStruct((_NW, _LANES), jnp.float32),
        mesh=mesh,
        scratch_types=[
            pltpu.VMEM((_NCH, _CH), jnp.int32),
            pltpu.VMEM((_CH, _D), jnp.float32),
            pltpu.VMEM((_CH, _D), jnp.float32),
            pltpu.VMEM((_CH, _D), jnp.float32),
            pltpu.VMEM((_CH, _D), jnp.float32),
            pltpu.VMEM((_LANES,), jnp.float32),
            pltpu.SemaphoreType.DMA,
            pltpu.SemaphoreType.DMA,
            pltpu.SemaphoreType.DMA,
            pltpu.SemaphoreType.DMA,
            pltpu.SemaphoreType.DMA,
        ],
    )(_sc_body)
    return run(latent, labels3d, centers)


def kernel(latent, labels, centers):
    labels3d = labels.astype(jnp.int32).reshape(_NW, _NCH, _CH)
    partials = _center_loss_partials(latent, labels3d, centers)
    return jnp.sum(partials) / jnp.float32(_B)


# speedup vs baseline: 2.0361x; 1.0001x over previous
import functools

import jax
import jax.numpy as jnp
from jax import lax
from jax.experimental import pallas as pl
from jax.experimental.pallas import tpu as pltpu
from jax.experimental.pallas import tpu_sc as plsc

_B = 16384
_D = 128
_NC = 2
_NS = 16
_NW = _NC * _NS
_RPW = _B // _NW
_CH = 128
_NCH = _RPW // _CH
_LANES = 16
_JV = _D // _LANES


def _sc_body(latent_hbm, labels_hbm, centers_hbm, out_hbm,
             lab_v, lat_v0, lat_v1, cen_v0, cen_v1, res_v,
             lat_sem0, lat_sem1, gat_sem0, gat_sem1, out_sem):
    wid = lax.axis_index("s") * _NC + lax.axis_index("c")
    pltpu.sync_copy(labels_hbm.at[wid], lab_v)

    lat_bufs = (lat_v0, lat_v1)
    cen_bufs = (cen_v0, cen_v1)
    lat_sems = (lat_sem0, lat_sem1)
    gat_sems = (gat_sem0, gat_sem1)

    def start(ch):
        b = ch % 2
        row0 = wid * _RPW + ch * _CH
        lat = pltpu.async_copy(
            latent_hbm.at[pl.ds(row0, _CH)], lat_bufs[b], lat_sems[b])
        gat = pltpu.async_copy(
            centers_hbm.at[lab_v.at[ch]], cen_bufs[b], gat_sems[b])
        return lat, gat

    accs = tuple(jnp.zeros((_LANES,), jnp.float32) for _ in range(_JV))
    pend = start(0)
    for ch in range(_NCH):
        b = ch % 2
        lat_copy, gat_copy = pend
        if ch + 1 < _NCH:
            pend = start(ch + 1)
        lat_copy.wait()
        gat_copy.wait()
        lat_v, cen_v = lat_bufs[b], cen_bufs[b]

        @plsc.parallel_loop(0, _CH, 1, unroll=4, carry=accs)
        def row_loop(r, acc_in):
            new = []
            for j in range(_JV):
                lt = lat_v[r, pl.ds(j * _LANES, _LANES)]
                cn = cen_v[r, pl.ds(j * _LANES, _LANES)]
                d = lt - cn
                new.append(acc_in[j] + d * d)
            return tuple(new)

        accs = row_loop

    total = accs[0]
    for j in range(1, _JV):
        total = total + accs[j]
    res_v[...] = total
    pltpu.async_copy(res_v, out_hbm.at[wid], out_sem).wait()


@jax.jit
def _center_loss_partials(latent, labels3d, centers):
    mesh = plsc.VectorSubcoreMesh(core_axis_name="c", subcore_axis_name="s")
    run = functools.partial(
        pl.kernel,
        out_type=jax.ShapeDtypeStruct((_NW, _LANES), jnp.float32),
        mesh=mesh,
        scratch_types=[
            pltpu.VMEM((_NCH, _CH), jnp.int32),
            pltpu.VMEM((_CH, _D), jnp.float32),
            pltpu.VMEM((_CH, _D), jnp.float32),
            pltpu.VMEM((_CH, _D), jnp.float32),
            pltpu.VMEM((_CH, _D), jnp.float32),
            pltpu.VMEM((_LANES,), jnp.float32),
            pltpu.SemaphoreType.DMA,
            pltpu.SemaphoreType.DMA,
            pltpu.SemaphoreType.DMA,
            pltpu.SemaphoreType.DMA,
            pltpu.SemaphoreType.DMA,
        ],
    )(_sc_body)
    return run(latent, labels3d, centers)


def kernel(latent, labels, centers):
    labels3d = labels.astype(jnp.int32).reshape(_NW, _NCH, _CH)
    partials = _center_loss_partials(latent, labels3d, centers)
    return jnp.sum(partials) / jnp.float32(_B)


# speedup vs baseline: 2.1599x; 1.0608x over previous
import functools

import jax
import jax.numpy as jnp
from jax import lax
from jax.experimental import pallas as pl
from jax.experimental.pallas import tpu as pltpu
from jax.experimental.pallas import tpu_sc as plsc

_B = 16384
_D = 128
_C = 1000
_NC = 2
_NS = 16
_NW = _NC * _NS
_RPW = _B // _NW
_CH = 128
_NCH = _RPW // _CH
_LANES = 16
_JV = _D // _LANES


def _sc_body(latent_hbm, labels_hbm, centers_hbm, out_hbm,
             lab_v, lat_v0, lat_v1, cen_v0, cen_v1, res_v, cen_sh,
             lat_sem0, lat_sem1, gat_sem0, gat_sem1, out_sem):
    sid = lax.axis_index("s")
    wid = sid * _NC + lax.axis_index("c")
    pltpu.sync_copy(labels_hbm.at[wid], lab_v)

    @pl.when(sid == 0)
    def _stage_table():
        pltpu.sync_copy(centers_hbm, cen_sh)

    lat_bufs = (lat_v0, lat_v1)
    cen_bufs = (cen_v0, cen_v1)
    lat_sems = (lat_sem0, lat_sem1)
    gat_sems = (gat_sem0, gat_sem1)

    def start(ch, first=False):
        b = ch % 2
        row0 = wid * _RPW + ch * _CH
        lat = pltpu.async_copy(
            latent_hbm.at[pl.ds(row0, _CH)], lat_bufs[b], lat_sems[b])
        if first:
            plsc.subcore_barrier()
        gat = pltpu.async_copy(
            cen_sh.at[lab_v.at[ch]], cen_bufs[b], gat_sems[b])
        return lat, gat

    accs = tuple(jnp.zeros((_LANES,), jnp.float32) for _ in range(_JV))
    pend = start(0, first=True)
    for ch in range(_NCH):
        b = ch % 2
        lat_copy, gat_copy = pend
        if ch + 1 < _NCH:
            pend = start(ch + 1)
        lat_copy.wait()
        gat_copy.wait()
        lat_v, cen_v = lat_bufs[b], cen_bufs[b]

        @plsc.parallel_loop(0, _CH, 1, unroll=4, carry=accs)
        def row_loop(r, acc_in):
            new = []
            for j in range(_JV):
                lt = lat_v[r, pl.ds(j * _LANES, _LANES)]
                cn = cen_v[r, pl.ds(j * _LANES, _LANES)]
                d = lt - cn
                new.append(acc_in[j] + d * d)
            return tuple(new)

        accs = row_loop

    total = accs[0]
    for j in range(1, _JV):
        total = total + accs[j]
    res_v[...] = total
    pltpu.async_copy(res_v, out_hbm.at[wid], out_sem).wait()


@jax.jit
def _center_loss_partials(latent, labels3d, centers):
    mesh = plsc.VectorSubcoreMesh(core_axis_name="c", subcore_axis_name="s")
    run = functools.partial(
        pl.kernel,
        out_type=jax.ShapeDtypeStruct((_NW, _LANES), jnp.float32),
        mesh=mesh,
        scratch_types=[
            pltpu.VMEM((_NCH, _CH), jnp.int32),
            pltpu.VMEM((_CH, _D), jnp.float32),
            pltpu.VMEM((_CH, _D), jnp.float32),
            pltpu.VMEM((_CH, _D), jnp.float32),
            pltpu.VMEM((_CH, _D), jnp.float32),
            pltpu.VMEM((_LANES,), jnp.float32),
            pltpu.VMEM_SHARED((_C, _D), jnp.float32),
            pltpu.SemaphoreType.DMA,
            pltpu.SemaphoreType.DMA,
            pltpu.SemaphoreType.DMA,
            pltpu.SemaphoreType.DMA,
            pltpu.SemaphoreType.DMA,
        ],
    )(_sc_body)
    return run(latent, labels3d, centers)


def kernel(latent, labels, centers):
    labels3d = labels.astype(jnp.int32).reshape(_NW, _NCH, _CH)
    partials = _center_loss_partials(latent, labels3d, centers)
    return jnp.sum(partials) / jnp.float32(_B)
